# baseline (speedup 1.0000x reference)
import functools

import jax
import jax.numpy as jnp
from jax import lax
from jax.experimental import pallas as pl
from jax.experimental.pallas import tpu as pltpu
from jax.experimental.pallas import tpu_sc as plsc

_D = 64
_B = 4096 * 50
_NC = 2
_NS = 16
_NW = _NC * _NS
_BPW = _B // _NW
_CHUNK = 128
_NSTEP = _BPW // _CHUNK
_NBUF = 10
_NGRP = _NSTEP // _NBUF

_mesh = plsc.VectorSubcoreMesh(core_axis_name="c", subcore_axis_name="s")


@functools.partial(
    pl.kernel,
    mesh=_mesh,
    out_type=jax.ShapeDtypeStruct((_B, _D), jnp.float32),
    scratch_types=[
        pltpu.VMEM((_BPW,), jnp.int32),
        pltpu.VMEM((_NBUF, _CHUNK, _D), jnp.float32),
    ]
    + [pltpu.SemaphoreType.DMA] * (2 * _NBUF),
    compiler_params=pltpu.CompilerParams(use_tc_tiling_on_sc=False),
)
def _gather_kernel(table, idx, out, idx_v, rows_v, *sems):
    gsem = sems[:_NBUF]
    osem = sems[_NBUF:]
    wid = lax.axis_index("s") * _NC + lax.axis_index("c")
    base = wid * _BPW
    pltpu.sync_copy(idx.at[pl.ds(base, _BPW)], idx_v)

    def g_copy(s, b):
        return pltpu.make_async_copy(
            table.at[idx_v.at[pl.ds(s * _CHUNK, _CHUNK)]], rows_v.at[b], gsem[b])

    def o_copy(s, b):
        return pltpu.make_async_copy(
            rows_v.at[b], out.at[pl.ds(base + s * _CHUNK, _CHUNK)], osem[b])

    for b in range(_NBUF):
        g_copy(b, b).start()

    def group(g, carry):
        for b in range(_NBUF):
            s = g * _NBUF + b
            g_copy(s, b).wait()
            o_copy(s, b).start()
            b1 = (b - 1) % _NBUF
            s_prev = s - 1
            s_next = s + _NBUF - 1

            @pl.when(jnp.logical_and(s_prev >= 0, s_next < _NSTEP))
            def _():
                o_copy(s_prev, b1).wait()
                g_copy(s_next, b1).start()

        return carry

    lax.fori_loop(0, _NGRP, group, 0)

    for k in range(_NBUF):
        s = _NSTEP - _NBUF + k
        o_copy(s, s % _NBUF).wait()


def kernel(input_ids, embedding):
    idx = input_ids.reshape(-1).astype(jnp.int32)
    out = _gather_kernel(embedding, idx)
    return out.reshape(input_ids.shape + (_D,))
